# Initial kernel scaffold; baseline (speedup 1.0000x reference)
import functools

import jax
import jax.numpy as jnp
from jax import lax
from jax.experimental import pallas as pl
from jax.experimental.pallas import tpu as pltpu
from jax.experimental.pallas import tpu_sc as plsc

N_NODES = 3_000_000
E_EDGES = 3_200_000
NGRAM = 5

CN = 1_536_000
DUMP = 8_192
ACC = CN + DUMP
ZSLICE = ACC // 16
BLK = 64
NBLK = E_EDGES // (BLK * 16)
WB = CN // 16
TAIL = N_NODES - CN - 15 * WB

ROWS2D = 25_000
DBLK = 1_000


def _data_body(f_ref, w_ref, v_ref, o_ref):
    prod = lax.dot_general(
        f_ref[...], w_ref[...], (((1,), (0,)), ((), ())),
        precision=lax.Precision.HIGHEST,
        preferred_element_type=jnp.float32)
    o_ref[...] = prod * v_ref[...]


def _compute_data(features, values, a0_weight):
    f2d = features.reshape(ROWS2D, 128 * NGRAM)
    v2d = values.reshape(ROWS2D, 128)
    w = jnp.kron(jnp.eye(128, dtype=jnp.float32), a0_weight.reshape(NGRAM, 1))
    out = pl.pallas_call(
        _data_body,
        grid=(ROWS2D // DBLK,),
        in_specs=[
            pl.BlockSpec((DBLK, 128 * NGRAM), lambda i: (i, 0)),
            pl.BlockSpec((128 * NGRAM, 128), lambda i: (0, 0)),
            pl.BlockSpec((DBLK, 128), lambda i: (i, 0)),
        ],
        out_specs=pl.BlockSpec((DBLK, 128), lambda i: (i, 0)),
        out_shape=jax.ShapeDtypeStruct((ROWS2D, 128), jnp.float32),
    )(f2d, w, v2d)
    return out.reshape(E_EDGES)


def _sc_scatter_body(rows_hbm, cols_hbm, data_hbm, zeros_hbm,
                     mem0_hbm, mem1_hbm, acc, nbuf, dbuf, ibuf):
    c = lax.axis_index("c")
    s = lax.axis_index("s")
    q, rem = divmod(NBLK, 16)
    nb = jnp.where(s < rem, q + 1, q)
    start = s * q + jnp.minimum(s, rem)

    def scan(src_hbm, p):
        def body(i, carry):
            base = (start + i) * BLK
            pltpu.sync_copy(src_hbm.at[pl.ds(base, BLK)], nbuf)
            pltpu.sync_copy(data_hbm.at[pl.ds(base, BLK)], dbuf)
            for j in range(BLK):
                r = nbuf[j]
                if p == 0:
                    sel = r < CN
                    loc = r
                else:
                    sel = r >= CN
                    loc = r - CN
                ibuf[j] = jnp.where(sel, loc,
                                    CN + jnp.bitwise_and(r, DUMP - 1))
            pltpu.sync_copy(dbuf, acc.at[ibuf], add=True)
            return carry
        lax.fori_loop(0, nb, body, 0)

    for p in range(2):
        pltpu.sync_copy(zeros_hbm, acc.at[pl.ds(s * ZSLICE, ZSLICE)])
        plsc.subcore_barrier()

        @pl.when(c == 0)
        def _():
            scan(rows_hbm, p)

        @pl.when(c == 1)
        def _():
            scan(cols_hbm, p)

        plsc.subcore_barrier()

        if p == 0:
            @pl.when(c == 0)
            def _():
                pltpu.sync_copy(acc.at[pl.ds(s * WB, WB)],
                                mem0_hbm.at[pl.ds(s * WB, WB)])

            @pl.when(c == 1)
            def _():
                pltpu.sync_copy(acc.at[pl.ds(s * WB, WB)],
                                mem1_hbm.at[pl.ds(s * WB, WB)])
        else:
            @pl.when(jnp.logical_and(c == 0, s < 15))
            def _():
                pltpu.sync_copy(acc.at[pl.ds(s * WB, WB)],
                                mem0_hbm.at[pl.ds(CN + s * WB, WB)])

            @pl.when(jnp.logical_and(c == 1, s < 15))
            def _():
                pltpu.sync_copy(acc.at[pl.ds(s * WB, WB)],
                                mem1_hbm.at[pl.ds(CN + s * WB, WB)])

            @pl.when(jnp.logical_and(c == 0, s == 15))
            def _():
                pltpu.sync_copy(acc.at[pl.ds(15 * WB, TAIL)],
                                mem0_hbm.at[pl.ds(CN + 15 * WB, TAIL)])

            @pl.when(jnp.logical_and(c == 1, s == 15))
            def _():
                pltpu.sync_copy(acc.at[pl.ds(15 * WB, TAIL)],
                                mem1_hbm.at[pl.ds(CN + 15 * WB, TAIL)])

        plsc.subcore_barrier()


_sc_scatter = functools.partial(
    pl.kernel,
    out_type=(jax.ShapeDtypeStruct((N_NODES,), jnp.float32),
              jax.ShapeDtypeStruct((N_NODES,), jnp.float32)),
    mesh=plsc.VectorSubcoreMesh(core_axis_name="c", subcore_axis_name="s"),
    scratch_types=[
        pltpu.VMEM_SHARED((ACC,), jnp.float32),
        pltpu.VMEM((BLK, 16), jnp.int32),
        pltpu.VMEM((BLK, 16), jnp.float32),
        pltpu.VMEM((BLK, 16), jnp.int32),
    ],
)(_sc_scatter_body)


def kernel(values, features, rows, cols, a0_weight):
    data = _compute_data(features, values, a0_weight)
    rows2d = rows.reshape(E_EDGES // 16, 16)
    cols2d = cols.reshape(E_EDGES // 16, 16)
    data2d = data.reshape(E_EDGES // 16, 16)
    zeros = jnp.zeros((ZSLICE,), jnp.float32)
    mem0, mem1 = _sc_scatter(rows2d, cols2d, data2d, zeros)
    indices = jnp.stack([rows, cols])
    mem = jnp.stack([mem0, mem1], axis=1)
    return (indices, data, mem)


# speedup vs baseline: 7.7337x; 7.7337x over previous
import functools

import jax
import jax.numpy as jnp
from jax import lax
from jax.experimental import pallas as pl
from jax.experimental.pallas import tpu as pltpu
from jax.experimental.pallas import tpu_sc as plsc

N_NODES = 3_000_000
E_EDGES = 3_200_000
NGRAM = 5

CN = 1_536_000
DUMP = 8_192
ACC = CN + DUMP
ZSLICE = ACC // 16
BLK = 64
NBLK = E_EDGES // (BLK * 16)
WB = CN // 16
TAIL = N_NODES - CN - 15 * WB

ROWS2D = 25_000
DBLK = 1_000


def _data_body(f_ref, w_ref, v_ref, o_ref):
    prod = lax.dot_general(
        f_ref[...], w_ref[...], (((1,), (0,)), ((), ())),
        precision=lax.Precision.HIGHEST,
        preferred_element_type=jnp.float32)
    o_ref[...] = prod * v_ref[...]


def _compute_data(features, values, a0_weight):
    f2d = features.reshape(ROWS2D, 128 * NGRAM)
    v2d = values.reshape(ROWS2D, 128)
    w = jnp.kron(jnp.eye(128, dtype=jnp.float32), a0_weight.reshape(NGRAM, 1))
    out = pl.pallas_call(
        _data_body,
        grid=(ROWS2D // DBLK,),
        in_specs=[
            pl.BlockSpec((DBLK, 128 * NGRAM), lambda i: (i, 0)),
            pl.BlockSpec((128 * NGRAM, 128), lambda i: (0, 0)),
            pl.BlockSpec((DBLK, 128), lambda i: (i, 0)),
        ],
        out_specs=pl.BlockSpec((DBLK, 128), lambda i: (i, 0)),
        out_shape=jax.ShapeDtypeStruct((ROWS2D, 128), jnp.float32),
    )(f2d, w, v2d)
    return out.reshape(E_EDGES)


def _sc_scatter_body(rows_hbm, cols_hbm, data_hbm, zeros_hbm,
                     mem0_hbm, mem1_hbm, acc, nbuf, dbuf, ibuf):
    c = lax.axis_index("c")
    s = lax.axis_index("s")
    q, rem = divmod(NBLK, 16)
    nb = jnp.where(s < rem, q + 1, q)
    start = s * q + jnp.minimum(s, rem)

    def scan(src_hbm, p):
        def body(i, carry):
            base = (start + i) * (BLK * 16)
            pltpu.sync_copy(src_hbm.at[pl.ds(base, BLK * 16)], nbuf)
            pltpu.sync_copy(data_hbm.at[pl.ds(base, BLK * 16)], dbuf)
            for j in range(BLK):
                r = nbuf[pl.ds(j * 16, 16)]
                if p == 0:
                    sel = r < CN
                    loc = r
                else:
                    sel = r >= CN
                    loc = r - CN
                ibuf[pl.ds(j * 16, 16)] = jnp.where(
                    sel, loc, CN + jnp.bitwise_and(r, DUMP - 1))
            pltpu.sync_copy(dbuf, acc.at[ibuf], add=True)
            return carry
        lax.fori_loop(0, nb, body, 0)

    for p in range(2):
        pltpu.sync_copy(zeros_hbm, acc.at[pl.ds(s * ZSLICE, ZSLICE)])
        plsc.subcore_barrier()

        @pl.when(c == 0)
        def _():
            scan(rows_hbm, p)

        @pl.when(c == 1)
        def _():
            scan(cols_hbm, p)

        plsc.subcore_barrier()

        @pl.when(c == 0)
        def _():
            pltpu.sync_copy(acc.at[pl.ds(s * WB, WB)],
                            mem0_hbm.at[pl.ds(p * CN + s * WB, WB)])

        @pl.when(c == 1)
        def _():
            pltpu.sync_copy(acc.at[pl.ds(s * WB, WB)],
                            mem1_hbm.at[pl.ds(p * CN + s * WB, WB)])

        plsc.subcore_barrier()


_sc_scatter = functools.partial(
    pl.kernel,
    out_type=(jax.ShapeDtypeStruct((2 * CN,), jnp.float32),
              jax.ShapeDtypeStruct((2 * CN,), jnp.float32)),
    mesh=plsc.VectorSubcoreMesh(core_axis_name="c", subcore_axis_name="s"),
    scratch_types=[
        pltpu.VMEM_SHARED((ACC,), jnp.float32),
        pltpu.VMEM((BLK * 16,), jnp.int32),
        pltpu.VMEM((BLK * 16,), jnp.float32),
        pltpu.VMEM((BLK * 16,), jnp.int32),
    ],
)(_sc_scatter_body)


def kernel(values, features, rows, cols, a0_weight):
    data = _compute_data(features, values, a0_weight)
    zeros = jnp.zeros((ZSLICE,), jnp.float32)
    mem0, mem1 = _sc_scatter(rows, cols, data, zeros)
    indices = jnp.stack([rows, cols])
    mem = jnp.stack([mem0[:N_NODES], mem1[:N_NODES]], axis=1)
    return (indices, data, mem)
